# Initial kernel scaffold; baseline (speedup 1.0000x reference)
import jax
import jax.numpy as jnp
from jax.experimental import pallas as pl


def kernel(edge_attr, node_feat_src, node_feat_dst, edge_index, W, b):
    raise NotImplementedError("write your pallas kernel here")


# speedup vs baseline: 5.4273x; 5.4273x over previous
import functools

import jax
import jax.numpy as jnp
from jax import lax
from jax.experimental import pallas as pl
from jax.experimental.pallas import tpu as pltpu
from jax.experimental.pallas import tpu_sc as plsc

N_NODES = 10000
N_EDGES = 320000
NODE_DIM = 128
EDGE_DIM = 16

_NC = 2
_NS = 16
_NW = _NC * _NS
_PER_W = N_EDGES // _NW
_CHUNK = 2000
_NCHUNKS = _PER_W // _CHUNK


def _proj_body(nsrc_ref, ndst_ref, ws_ref, wd_ref, b_ref, sp_ref, dp_ref):
    sp_ref[...] = (
        jnp.dot(nsrc_ref[...], ws_ref[...], preferred_element_type=jnp.float32)
        + b_ref[...]
    )
    dp_ref[...] = jnp.dot(
        ndst_ref[...], wd_ref[...], preferred_element_type=jnp.float32
    )


def _edge_body(ea_ref, g_ref, bd_ref, out_ref):
    out_ref[...] = (
        jnp.dot(ea_ref[...], bd_ref[...], preferred_element_type=jnp.float32)
        + g_ref[...]
    )


def _gather_sum_body(sp_hbm, dp_hbm, sidx_hbm, didx_hbm, out_hbm,
                     sidx_v, didx_v, srows_v, drows_v, sem_s, sem_d):
    wid = lax.axis_index("s") * _NC + lax.axis_index("c")
    base = wid * _PER_W
    for c in range(_NCHUNKS):
        off = base + c * _CHUNK
        pltpu.sync_copy(sidx_hbm.at[pl.ds(off, _CHUNK)], sidx_v)
        pltpu.sync_copy(didx_hbm.at[pl.ds(off, _CHUNK)], didx_v)
        cp_s = pltpu.async_copy(sp_hbm.at[sidx_v], srows_v, sem_s)
        cp_d = pltpu.async_copy(dp_hbm.at[didx_v], drows_v, sem_d)
        cp_s.wait()
        cp_d.wait()

        @plsc.parallel_loop(0, _CHUNK, 1, unroll=8)
        def _add(i):
            srows_v[i] = srows_v[i] + drows_v[i]

        pltpu.sync_copy(srows_v, out_hbm.at[pl.ds(off, _CHUNK)])


def kernel(edge_attr, node_feat_src, node_feat_dst, edge_index, W, b):
    src_idx = edge_index[0].astype(jnp.int32)
    dst_idx = edge_index[1].astype(jnp.int32)
    W_e = W[:EDGE_DIM]
    W_s = W[EDGE_DIM:EDGE_DIM + NODE_DIM]
    W_d = W[EDGE_DIM + NODE_DIM:]

    rows_blk = 2000
    grid1 = N_NODES // rows_blk
    src_proj, dst_proj = pl.pallas_call(
        _proj_body,
        grid=(grid1,),
        in_specs=[
            pl.BlockSpec((rows_blk, NODE_DIM), lambda i: (i, 0)),
            pl.BlockSpec((rows_blk, NODE_DIM), lambda i: (i, 0)),
            pl.BlockSpec((NODE_DIM, EDGE_DIM), lambda i: (0, 0)),
            pl.BlockSpec((NODE_DIM, EDGE_DIM), lambda i: (0, 0)),
            pl.BlockSpec((1, EDGE_DIM), lambda i: (0, 0)),
        ],
        out_specs=[
            pl.BlockSpec((rows_blk, EDGE_DIM), lambda i: (i, 0)),
            pl.BlockSpec((rows_blk, EDGE_DIM), lambda i: (i, 0)),
        ],
        out_shape=[
            jax.ShapeDtypeStruct((N_NODES, EDGE_DIM), jnp.float32),
            jax.ShapeDtypeStruct((N_NODES, EDGE_DIM), jnp.float32),
        ],
    )(node_feat_src, node_feat_dst, W_s, W_d, b.reshape(1, EDGE_DIM))

    mesh = plsc.VectorSubcoreMesh(core_axis_name="c", subcore_axis_name="s")
    g = pl.kernel(
        _gather_sum_body,
        out_type=jax.ShapeDtypeStruct((N_EDGES, EDGE_DIM), jnp.float32),
        mesh=mesh,
        scratch_types=[
            pltpu.VMEM((_CHUNK,), jnp.int32),
            pltpu.VMEM((_CHUNK,), jnp.int32),
            pltpu.VMEM((_CHUNK, EDGE_DIM), jnp.float32),
            pltpu.VMEM((_CHUNK, EDGE_DIM), jnp.float32),
            pltpu.SemaphoreType.DMA,
            pltpu.SemaphoreType.DMA,
        ],
        compiler_params=pltpu.CompilerParams(use_tc_tiling_on_sc=False),
    )(src_proj, dst_proj, src_idx, dst_idx)

    M = jnp.eye(EDGE_DIM, dtype=jnp.float32) + W_e
    BD = jnp.kron(jnp.eye(8, dtype=jnp.float32), M)
    pack = 128 // EDGE_DIM
    rows2 = N_EDGES // pack
    ea_r = edge_attr.reshape(rows2, 128)
    g_r = g.reshape(rows2, 128)
    blk2 = 5000
    grid2 = rows2 // blk2
    out_r = pl.pallas_call(
        _edge_body,
        grid=(grid2,),
        in_specs=[
            pl.BlockSpec((blk2, 128), lambda i: (i, 0)),
            pl.BlockSpec((blk2, 128), lambda i: (i, 0)),
            pl.BlockSpec((128, 128), lambda i: (0, 0)),
        ],
        out_specs=pl.BlockSpec((blk2, 128), lambda i: (i, 0)),
        out_shape=jax.ShapeDtypeStruct((rows2, 128), jnp.float32),
    )(ea_r, g_r, BD)
    return out_r.reshape(N_EDGES, EDGE_DIM)
